# baseline (speedup 1.0000x reference)
import functools

import jax
import jax.numpy as jnp
from jax import lax
from jax.experimental import pallas as pl
from jax.experimental.pallas import tpu as pltpu

TK1 = 2
TK2 = 2


def _top2_cols(logits, n):
    iota = lax.broadcasted_iota(jnp.int32, logits.shape, 0)
    m1 = jnp.max(logits, axis=0, keepdims=True)
    i1 = jnp.min(jnp.where(logits == m1, iota, n), axis=0, keepdims=True)
    masked = jnp.where(iota == i1, -jnp.inf, logits)
    m2 = jnp.max(masked, axis=0, keepdims=True)
    i2 = jnp.min(jnp.where(masked == m2, iota, n), axis=0, keepdims=True)
    d = jnp.exp(m2 - m1)
    w1 = 1.0 / (1.0 + d)
    w2 = d / (1.0 + d)
    return i1, i2, w1, w2


def _sel(onehot, rows):
    return jnp.sum(onehot * rows, axis=0, keepdims=True)


def _routing_kernel(xt_ref, gw_ref, ggw_ref,
                    wdt_ref, rt_ref, cnt_ref,
                    *, G, E, CH):
    T = xt_ref.shape[0]
    NEXP = G * E
    xt = xt_ref[...]

    gl = lax.dot_general(gw_ref[...], xt, (((1,), (1,)), ((), ())),
                         preferred_element_type=jnp.float32)
    g1, g2, gw1, gw2 = _top2_cols(gl, G)

    el = lax.dot_general(ggw_ref[...], xt, (((1,), (1,)), ((), ())),
                         preferred_element_type=jnp.float32)
    e1s, e2s, we1s, we2s = [], [], [], []
    for g in range(G):
        a, b, wa, wb = _top2_cols(el[g * E:(g + 1) * E, :], E)
        e1s.append(a); e2s.append(b); we1s.append(wa); we2s.append(wb)
    E1 = jnp.concatenate(e1s, axis=0)
    E2 = jnp.concatenate(e2s, axis=0)
    WE1 = jnp.concatenate(we1s, axis=0)
    WE2 = jnp.concatenate(we2s, axis=0)

    iota_g = lax.broadcasted_iota(jnp.int32, (G, T), 0)
    oh1 = (iota_g == g1).astype(jnp.float32)
    oh2 = (iota_g == g2).astype(jnp.float32)
    sel_e = [g1 * E + _sel(oh1, E1.astype(jnp.float32)).astype(jnp.int32),
             g1 * E + _sel(oh1, E2.astype(jnp.float32)).astype(jnp.int32),
             g2 * E + _sel(oh2, E1.astype(jnp.float32)).astype(jnp.int32),
             g2 * E + _sel(oh2, E2.astype(jnp.float32)).astype(jnp.int32)]
    sel_w = [gw1 * _sel(oh1, WE1), gw1 * _sel(oh1, WE2),
             gw2 * _sel(oh2, WE1), gw2 * _sel(oh2, WE2)]

    iota_e = lax.broadcasted_iota(jnp.int32, (NEXP, T), 0)
    nz = jnp.zeros((NEXP, T), dtype=jnp.float32)
    wdt = jnp.zeros((NEXP, T), dtype=jnp.float32)
    for k in range(4):
        hit = (iota_e == sel_e[k]).astype(jnp.float32)
        nz = nz + hit
        wdt = wdt + hit * sel_w[k]
    wdt_ref[...] = wdt

    ri = lax.broadcasted_iota(jnp.int32, (CH, CH), 0)
    ci = lax.broadcasted_iota(jnp.int32, (CH, CH), 1)
    triu = (ri <= ci).astype(jnp.float32)
    carry = jnp.zeros((NEXP, 1), dtype=jnp.float32)
    for c in range(T // CH):
        chunk = nz[:, c * CH:(c + 1) * CH]
        incl = jnp.dot(chunk, triu, preferred_element_type=jnp.float32)
        excl = incl - chunk + carry
        rt_ref[:, c * CH:(c + 1) * CH] = jnp.where(
            chunk > 0.5, excl.astype(jnp.int32), -1)
        carry = carry + incl[:, CH - 1:CH]

    cnt_ref[...] = carry.astype(jnp.int32).reshape(1, NEXP)


def _ffn_kernel(cnt_ref,
                xt_ref, rt_ref, wdt_ref, w1_ref, b1_ref, w2_ref, b2_ref,
                out_ref, *, BK):
    e = pl.program_id(0)

    @pl.when(e == 0)
    def _init():
        out_ref[...] = jnp.zeros_like(out_ref)

    nblk = lax.div(cnt_ref[e] + (BK - 1), BK)

    def _block(i, carry_unused):
        target0 = i * BK
        rrow = rt_ref[0]
        iota_r = lax.broadcasted_iota(jnp.int32, (BK, 1), 0)
        hit = rrow == iota_r + target0
        pf = hit.astype(jnp.bfloat16)
        pfw = pf * wdt_ref[0].astype(jnp.bfloat16)
        xb = jnp.dot(pf, xt_ref[...], preferred_element_type=jnp.float32)
        h = jnp.maximum(
            lax.dot_general(xb.astype(jnp.bfloat16),
                            w1_ref[0].astype(jnp.bfloat16),
                            (((1,), (1,)), ((), ())),
                            preferred_element_type=jnp.float32)
            + b1_ref[0], 0.0)
        y = lax.dot_general(h.astype(jnp.bfloat16),
                            w2_ref[0].astype(jnp.bfloat16),
                            (((1,), (1,)), ((), ())),
                            preferred_element_type=jnp.float32) + b2_ref[0]
        out_ref[...] += lax.dot_general(
            pfw, y.astype(jnp.bfloat16), (((0,), (0,)), ((), ())),
            preferred_element_type=jnp.float32)
        return carry_unused

    lax.fori_loop(0, nblk, _block, 0)


def kernel(x, gate_w, group_gate_w, W1, b1, W2, b2):
    Bsz, S, D = x.shape
    G, E = group_gate_w.shape[:2]
    H = W1.shape[2]
    NEXP = G * E
    T = Bsz * S
    BK = min(128, T)
    CH = min(256, T)

    xt = x.reshape(T, D)
    ggw = group_gate_w.reshape(NEXP, D)
    W1r = W1.reshape(NEXP, H, D)
    W2r = W2.reshape(NEXP, D, H)
    b1r = b1.reshape(NEXP, H)
    b2r = b2.reshape(NEXP, D)

    wdt, rt, cnt = pl.pallas_call(
        functools.partial(_routing_kernel, G=G, E=E, CH=CH),
        out_shape=[
            jax.ShapeDtypeStruct((NEXP, T), jnp.float32),
            jax.ShapeDtypeStruct((NEXP, T), jnp.int32),
            jax.ShapeDtypeStruct((1, NEXP), jnp.int32),
        ],
    )(xt, gate_w, ggw)

    out = pl.pallas_call(
        functools.partial(_ffn_kernel, BK=BK),
        grid_spec=pltpu.PrefetchScalarGridSpec(
            num_scalar_prefetch=1,
            grid=(NEXP,),
            in_specs=[
                pl.BlockSpec((T, D), lambda e, c: (0, 0)),
                pl.BlockSpec((1, 1, T), lambda e, c: (e, 0, 0)),
                pl.BlockSpec((1, 1, T), lambda e, c: (e, 0, 0)),
                pl.BlockSpec((1, H, D), lambda e, c: (e, 0, 0)),
                pl.BlockSpec((1, 1, H), lambda e, c: (e, 0, 0)),
                pl.BlockSpec((1, D, H), lambda e, c: (e, 0, 0)),
                pl.BlockSpec((1, 1, D), lambda e, c: (e, 0, 0)),
            ],
            out_specs=pl.BlockSpec((T, D), lambda e, c: (0, 0)),
        ),
        out_shape=jax.ShapeDtypeStruct((T, D), jnp.float32),
        compiler_params=pltpu.CompilerParams(
            dimension_semantics=("arbitrary",)),
    )(cnt.reshape(NEXP),
      xt.astype(jnp.bfloat16), rt.reshape(NEXP, 1, T),
      wdt.reshape(NEXP, 1, T),
      W1r, b1r.reshape(NEXP, 1, H), W2r, b2r.reshape(NEXP, 1, D))

    return out.reshape(Bsz, S, D)


# speedup vs baseline: 1.3069x; 1.3069x over previous
import functools

import jax
import jax.numpy as jnp
from jax import lax
from jax.experimental import pallas as pl
from jax.experimental.pallas import tpu as pltpu

TK1 = 2
TK2 = 2


def _top2_cols(logits, n):
    iota = lax.broadcasted_iota(jnp.int32, logits.shape, 0)
    m1 = jnp.max(logits, axis=0, keepdims=True)
    i1 = jnp.min(jnp.where(logits == m1, iota, n), axis=0, keepdims=True)
    masked = jnp.where(iota == i1, -jnp.inf, logits)
    m2 = jnp.max(masked, axis=0, keepdims=True)
    i2 = jnp.min(jnp.where(masked == m2, iota, n), axis=0, keepdims=True)
    d = jnp.exp(m2 - m1)
    w1 = 1.0 / (1.0 + d)
    w2 = d / (1.0 + d)
    return i1, i2, w1, w2


def _sel(onehot, rows):
    return jnp.sum(onehot * rows, axis=0, keepdims=True)


def _routing_kernel(xt_ref, gw_ref, ggw_ref,
                    wdt_ref, rt_ref, cnt_ref,
                    *, G, E, CH):
    T = xt_ref.shape[0]
    NEXP = G * E
    xt = xt_ref[...]

    gl = lax.dot_general(gw_ref[...], xt, (((1,), (1,)), ((), ())),
                         preferred_element_type=jnp.float32)
    g1, g2, gw1, gw2 = _top2_cols(gl, G)

    el = lax.dot_general(ggw_ref[...], xt, (((1,), (1,)), ((), ())),
                         preferred_element_type=jnp.float32)
    e1s, e2s, we1s, we2s = [], [], [], []
    for g in range(G):
        a, b, wa, wb = _top2_cols(el[g * E:(g + 1) * E, :], E)
        e1s.append(a); e2s.append(b); we1s.append(wa); we2s.append(wb)
    E1 = jnp.concatenate(e1s, axis=0)
    E2 = jnp.concatenate(e2s, axis=0)
    WE1 = jnp.concatenate(we1s, axis=0)
    WE2 = jnp.concatenate(we2s, axis=0)

    iota_g = lax.broadcasted_iota(jnp.int32, (G, T), 0)
    oh1 = (iota_g == g1).astype(jnp.float32)
    oh2 = (iota_g == g2).astype(jnp.float32)
    sel_e = [g1 * E + _sel(oh1, E1.astype(jnp.float32)).astype(jnp.int32),
             g1 * E + _sel(oh1, E2.astype(jnp.float32)).astype(jnp.int32),
             g2 * E + _sel(oh2, E1.astype(jnp.float32)).astype(jnp.int32),
             g2 * E + _sel(oh2, E2.astype(jnp.float32)).astype(jnp.int32)]
    sel_w = [gw1 * _sel(oh1, WE1), gw1 * _sel(oh1, WE2),
             gw2 * _sel(oh2, WE1), gw2 * _sel(oh2, WE2)]

    iota_e = lax.broadcasted_iota(jnp.int32, (NEXP, T), 0)
    nz = jnp.zeros((NEXP, T), dtype=jnp.float32)
    wdt = jnp.zeros((NEXP, T), dtype=jnp.float32)
    for k in range(4):
        hit = (iota_e == sel_e[k]).astype(jnp.float32)
        nz = nz + hit
        wdt = wdt + hit * sel_w[k]
    wdt_ref[...] = wdt

    ri = lax.broadcasted_iota(jnp.int32, (CH, CH), 0)
    ci = lax.broadcasted_iota(jnp.int32, (CH, CH), 1)
    triu = (ri <= ci).astype(jnp.float32)
    carry = jnp.zeros((NEXP, 1), dtype=jnp.float32)
    for c in range(T // CH):
        chunk = nz[:, c * CH:(c + 1) * CH]
        incl = jnp.dot(chunk, triu, preferred_element_type=jnp.float32)
        excl = incl - chunk + carry
        rt_ref[:, c * CH:(c + 1) * CH] = jnp.where(
            chunk > 0.5, excl.astype(jnp.int32), -1)
        carry = carry + incl[:, CH - 1:CH]

    cnt_ref[...] = carry.astype(jnp.int32).reshape(1, NEXP)


def _ffn_kernel(cnt_ref,
                xt_ref, rt_ref, wdt_ref, w1_ref, b1_ref, w2_ref, b2_ref,
                out_ref, *, BK, EPP):
    s = pl.program_id(0)

    @pl.when(s == 0)
    def _init():
        out_ref[...] = jnp.zeros_like(out_ref)

    nblk = lax.div(cnt_ref[s * EPP] + (BK - 1), BK)
    for j in range(1, EPP):
        nblk = jnp.maximum(nblk, lax.div(cnt_ref[s * EPP + j] + (BK - 1), BK))

    def _block(i, carry_unused):
        target0 = i * BK
        iota_r = lax.broadcasted_iota(jnp.int32, (BK, 1), 0)
        pfs, pfws, ys = [], [], []
        for j in range(EPP):
            rrow = rt_ref[j]
            hit = rrow == iota_r + target0
            pf = hit.astype(jnp.bfloat16)
            pfs.append(pf)
            pfws.append(pf * wdt_ref[j].astype(jnp.bfloat16))
        xb = jnp.dot(jnp.concatenate(pfs, axis=0), xt_ref[...],
                     preferred_element_type=jnp.float32)
        for j in range(EPP):
            h = jnp.maximum(
                lax.dot_general(
                    xb[j * BK:(j + 1) * BK].astype(jnp.bfloat16),
                    w1_ref[j].astype(jnp.bfloat16),
                    (((1,), (1,)), ((), ())),
                    preferred_element_type=jnp.float32)
                + b1_ref[j], 0.0)
            y = lax.dot_general(h.astype(jnp.bfloat16),
                                w2_ref[j].astype(jnp.bfloat16),
                                (((1,), (1,)), ((), ())),
                                preferred_element_type=jnp.float32) + b2_ref[j]
            ys.append(y.astype(jnp.bfloat16))
        out_ref[...] += lax.dot_general(
            jnp.concatenate(pfws, axis=0), jnp.concatenate(ys, axis=0),
            (((0,), (0,)), ((), ())),
            preferred_element_type=jnp.float32)
        return carry_unused

    lax.fori_loop(0, nblk, _block, 0)


def kernel(x, gate_w, group_gate_w, W1, b1, W2, b2):
    Bsz, S, D = x.shape
    G, E = group_gate_w.shape[:2]
    H = W1.shape[2]
    NEXP = G * E
    T = Bsz * S
    BK = min(192, T)
    CH = min(256, T)
    EPP = 2 if NEXP % 2 == 0 else 1

    xt = x.reshape(T, D)
    ggw = group_gate_w.reshape(NEXP, D)
    W1r = W1.reshape(NEXP, H, D)
    W2r = W2.reshape(NEXP, D, H)
    b1r = b1.reshape(NEXP, H)
    b2r = b2.reshape(NEXP, D)

    wdt, rt, cnt = pl.pallas_call(
        functools.partial(_routing_kernel, G=G, E=E, CH=CH),
        out_shape=[
            jax.ShapeDtypeStruct((NEXP, T), jnp.float32),
            jax.ShapeDtypeStruct((NEXP, T), jnp.int32),
            jax.ShapeDtypeStruct((1, NEXP), jnp.int32),
        ],
    )(xt, gate_w, ggw)

    out = pl.pallas_call(
        functools.partial(_ffn_kernel, BK=BK, EPP=EPP),
        grid_spec=pltpu.PrefetchScalarGridSpec(
            num_scalar_prefetch=1,
            grid=(NEXP // EPP,),
            in_specs=[
                pl.BlockSpec((T, D), lambda e, c: (0, 0)),
                pl.BlockSpec((EPP, 1, T), lambda e, c: (e, 0, 0)),
                pl.BlockSpec((EPP, 1, T), lambda e, c: (e, 0, 0)),
                pl.BlockSpec((EPP, H, D), lambda e, c: (e, 0, 0)),
                pl.BlockSpec((EPP, 1, H), lambda e, c: (e, 0, 0)),
                pl.BlockSpec((EPP, D, H), lambda e, c: (e, 0, 0)),
                pl.BlockSpec((EPP, 1, D), lambda e, c: (e, 0, 0)),
            ],
            out_specs=pl.BlockSpec((T, D), lambda e, c: (0, 0)),
        ),
        out_shape=jax.ShapeDtypeStruct((T, D), jnp.float32),
        compiler_params=pltpu.CompilerParams(
            dimension_semantics=("arbitrary",)),
    )(cnt.reshape(NEXP),
      xt.astype(jnp.bfloat16), rt.reshape(NEXP, 1, T),
      wdt.reshape(NEXP, 1, T),
      W1r, b1r.reshape(NEXP, 1, H), W2r, b2r.reshape(NEXP, 1, D))

    return out.reshape(Bsz, S, D)
